# baseline (speedup 1.0000x reference)
import jax
import jax.numpy as jnp
from jax import lax
from jax.experimental import pallas as pl
from jax.experimental.pallas import tpu as pltpu
from jax.experimental.pallas import tpu_sc as plsc

D = 64
L = 16
NC = 2
NS = 16
NW = NC * NS
B = 16384
NV = B // L
COLS = 7813
CPW = 489
RING = 8
RSTAGE = 8
SENT = 511


_DNUMS = lax.GatherDimensionNumbers(
    offset_dims=(), collapsed_slice_dims=(0,), start_index_map=(0,))


def _vgather(x, idx):
    return lax.gather(x, idx[:, None], _DNUMS, (1,),
                      mode=lax.GatherScatterMode.PROMISE_IN_BOUNDS)


def _dx(ref, t):
    v = ref[pl.ds((t >> 4) << 4, L)]
    g = _vgather(v, jnp.full((L,), t & 15, jnp.int32))
    return g[0]


def _phase(idx_hbm, T3, rows_hbm, idxv, hitbuf, counts, bases, bases2,
           touched, panels, rowstage, sem_panel, sem_row, c0, c1, iota):
    pltpu.sync_copy(idx_hbm, idxv)

    for j in range(32):
        counts[pl.ds(j * L, L)] = jnp.zeros((L,), jnp.int32)

    prev_idx = jnp.maximum(iota - 1, 0)
    next_idx = jnp.minimum(iota + 1, L - 1)

    def runs(lc_s):
        pv = _vgather(lc_s, prev_idx)
        nx = _vgather(lc_s, next_idx)
        isstart = (iota == 0) | (lc_s != pv)
        isend = (iota == L - 1) | (lc_s != nx)
        startidx = plsc.cummax(jnp.where(isstart, iota, 0))
        off = iota - startidx
        return off, isend

    def hist(j, carry):
        v = idxv[pl.ds(j * L, L)]
        col = v >> 7
        m = (col >= c0) & (col < c1)
        lc = jnp.where(m, col - c0, SENT)
        lc_s, _ = plsc.sort_key_val(lc, v)
        off, isend = runs(lc_s)
        plsc.addupdate_scatter(counts, [lc_s], off + 1,
                               mask=isend & (lc_s < SENT))
        return carry

    lax.fori_loop(0, NV, hist, 0, unroll=4)

    carry = jnp.int32(0)
    for j in range(32):
        cv = counts[pl.ds(j * L, L)]
        cs = plsc.cumsum(cv)
        ex = cs - cv + carry
        bases[pl.ds(j * L, L)] = ex
        bases2[pl.ds(j * L, L)] = ex
        carry = carry + cs[L - 1]

    def place(j, carry2):
        v = idxv[pl.ds(j * L, L)]
        pos = j * L + iota
        col = v >> 7
        m = (col >= c0) & (col < c1)
        lc = jnp.where(m, col - c0, SENT)
        lane7 = v & 127
        packed = pos | (lane7 << 14) | (lc << 21)
        lc_s, packed_s = plsc.sort_key_val(lc, packed)
        off, isend = runs(lc_s)
        mreal = lc_s < SENT
        base_g = plsc.load_gather(bases2, [lc_s])
        plsc.store_scatter(hitbuf, [base_g + off], packed_s, mask=mreal)
        plsc.addupdate_scatter(bases2, [lc_s], off + 1, mask=isend & mreal)
        return carry2

    lax.fori_loop(0, NV, place, 0, unroll=4)

    nt = jnp.int32(0)
    for j in range(32):
        cv = counts[pl.ds(j * L, L)]
        m = cv > 0
        mi = m.astype(jnp.int32)
        within = plsc.cumsum(mi)
        dest = nt + within - mi
        plsc.store_scatter(touched, [dest], j * L + iota, mask=m)
        nt = nt + within[L - 1]

    def issue(t):
        colid = _dx(touched, t)
        fb = (colid + c0) * 128
        slot = lax.rem(t, RING)
        src = T3.at[:, :, pl.ds(pl.multiple_of(fb, 128), 128)]
        pltpu.async_copy(src, panels.at[slot], sem_panel)

    for r in range(RING):
        @pl.when(r < nt)
        def _():
            issue(jnp.int32(r))

    trv = [(c * L + iota) >> 3 for c in range(D // L)]
    fsv = [(c * L + iota) & 7 for c in range(D // L)]

    def loop_t(t, nrows):
        pltpu.make_async_copy(
            T3.at[:, :, pl.ds(0, 128)], panels.at[0], sem_panel).wait()
        slot = lax.rem(t, RING)
        slotv = jnp.full((L,), slot, jnp.int32)
        colid = _dx(touched, t)
        segbase = _dx(bases, colid)
        segcnt = _dx(counts, colid)

        def hloop(h, nr):
            @pl.when(nr >= RSTAGE)
            def _():
                pltpu.make_async_copy(
                    rowstage.at[pl.ds(0, D)], rows_hbm.at[pl.ds(0, D)],
                    sem_row).wait()
            hit = _dx(hitbuf, segbase + h)
            pos = hit & 0x3FFF
            lane7 = (hit >> 14) & 127
            lv = jnp.full((L,), lane7, jnp.int32)
            rslot = lax.rem(nr, RSTAGE)
            for c in range(D // L):
                gv = plsc.load_gather(panels, [slotv, trv[c], fsv[c], lv])
                rowstage[pl.ds(rslot * D + c * L, L)] = gv
            pltpu.async_copy(rowstage.at[pl.ds(rslot * D, D)],
                             rows_hbm.at[pl.ds(pos * D, D)], sem_row)
            return nr + 1

        nrows = lax.fori_loop(0, segcnt, hloop, nrows)

        @pl.when(t + RING < nt)
        def _():
            issue(t + RING)
        return nrows

    nrows = lax.fori_loop(0, nt, loop_t, jnp.int32(0))

    def drain(h, c):
        pltpu.make_async_copy(
            rowstage.at[pl.ds(0, D)], rows_hbm.at[pl.ds(0, D)],
            sem_row).wait()
        return c

    lax.fori_loop(0, jnp.minimum(nrows, RSTAGE), drain, 0)


def _body1(u_hbm, i_hbm, UT_hbm, VT_hbm, urows_hbm, vrows_hbm,
           idxv, hitbuf, counts, bases, bases2, touched, panels, rowstage,
           sem_panel, sem_row):
    wid = lax.axis_index("s") * NC + lax.axis_index("c")
    iota = lax.iota(jnp.int32, L)
    U3 = UT_hbm.reshape(8, 8, UT_hbm.shape[1])
    V3 = VT_hbm.reshape(8, 8, VT_hbm.shape[1])
    half = wid // NS
    hw = wid % NS
    c0 = hw * CPW
    c1 = jnp.minimum(c0 + CPW, COLS)

    @pl.when(half == 0)
    def _():
        _phase(u_hbm, U3, urows_hbm, idxv, hitbuf, counts, bases, bases2,
               touched, panels, rowstage, sem_panel, sem_row, c0, c1, iota)

    @pl.when(half == 1)
    def _():
        _phase(i_hbm, V3, vrows_hbm, idxv, hitbuf, counts, bases, bases2,
               touched, panels, rowstage, sem_panel, sem_row, c0, c1, iota)


def _body2(urows_hbm, vrows_hbm, out_hbm, uv, vv, out_v, sem_u, sem_v):
    bpw = B // NW
    wid = lax.axis_index("s") * NC + lax.axis_index("c")
    base = wid * bpw
    cu = pltpu.async_copy(urows_hbm.at[pl.ds(base * D, bpw * D)], uv, sem_u)
    cv = pltpu.async_copy(vrows_hbm.at[pl.ds(base * D, bpw * D)], vv, sem_v)
    cu.wait()
    cv.wait()
    lane = lax.iota(jnp.int32, L)

    def group(g, carry):
        b0 = g * L
        acc = jnp.zeros((L,), jnp.float32)
        for k in range(L):
            o = (b0 + k) * D
            s = uv[pl.ds(o, L)] * vv[pl.ds(o, L)]
            for c in range(1, D // L):
                s = s + uv[pl.ds(o + c * L, L)] * vv[pl.ds(o + c * L, L)]
            acc = jnp.where(lane == k, jnp.sum(s), acc)
        out_v[pl.ds(b0, L)] = acc
        return carry

    lax.fori_loop(0, B // NW // L, group, 0)
    pltpu.sync_copy(out_v, out_hbm.at[pl.ds(base, bpw)])


def kernel(u, i, U_emb, V_emb):
    mesh = plsc.VectorSubcoreMesh(core_axis_name="c", subcore_axis_name="s")
    f1 = pl.kernel(
        _body1,
        out_type=(jax.ShapeDtypeStruct((B * D,), jnp.float32),
                  jax.ShapeDtypeStruct((B * D,), jnp.float32)),
        mesh=mesh,
        compiler_params=pltpu.CompilerParams(
            needs_layout_passes=False, use_tc_tiling_on_sc=True),
        scratch_types=[
            pltpu.VMEM((B,), jnp.int32),
            pltpu.VMEM((B,), jnp.int32),
            pltpu.VMEM((512,), jnp.int32),
            pltpu.VMEM((512,), jnp.int32),
            pltpu.VMEM((512,), jnp.int32),
            pltpu.VMEM((512,), jnp.int32),
            pltpu.VMEM((RING, 8, 8, 128), jnp.float32),
            pltpu.VMEM((RSTAGE * D,), jnp.float32),
            pltpu.SemaphoreType.DMA,
            pltpu.SemaphoreType.DMA,
        ],
    )
    urows, vrows = f1(u.astype(jnp.int32), i.astype(jnp.int32),
                      U_emb.T, V_emb.T)
    bpw = B // NW
    f2 = pl.kernel(
        _body2,
        out_type=jax.ShapeDtypeStruct((B,), jnp.float32),
        mesh=mesh,
        compiler_params=pltpu.CompilerParams(
            needs_layout_passes=False, use_tc_tiling_on_sc=False),
        scratch_types=[
            pltpu.VMEM((bpw * D,), jnp.float32),
            pltpu.VMEM((bpw * D,), jnp.float32),
            pltpu.VMEM((bpw,), jnp.float32),
            pltpu.SemaphoreType.DMA,
            pltpu.SemaphoreType.DMA,
        ],
    )
    return f2(urows, vrows)


# speedup vs baseline: 1.0019x; 1.0019x over previous
import jax
import jax.numpy as jnp
from jax import lax
from jax.experimental import pallas as pl
from jax.experimental.pallas import tpu as pltpu
from jax.experimental.pallas import tpu_sc as plsc

D = 64
L = 16
NC = 2
NS = 16
NW = NC * NS
B = 16384
NV = B // L
COLS = 7813
CPW = 489
RING = 8
RSTAGE = 32
SENT = 511


_DNUMS = lax.GatherDimensionNumbers(
    offset_dims=(), collapsed_slice_dims=(0,), start_index_map=(0,))


def _vgather(x, idx):
    return lax.gather(x, idx[:, None], _DNUMS, (1,),
                      mode=lax.GatherScatterMode.PROMISE_IN_BOUNDS)


def _dx(ref, t):
    v = ref[pl.ds((t >> 4) << 4, L)]
    g = _vgather(v, jnp.full((L,), t & 15, jnp.int32))
    return g[0]


def _phase(idx_hbm, T3, rows_hbm, idxv, hitbuf, counts, bases, bases2,
           touched, panels, rowstage, sem_panel, sem_row, c0, c1, iota):
    pltpu.sync_copy(idx_hbm, idxv)

    for j in range(32):
        counts[pl.ds(j * L, L)] = jnp.zeros((L,), jnp.int32)

    prev_idx = jnp.maximum(iota - 1, 0)
    next_idx = jnp.minimum(iota + 1, L - 1)

    def runs(lc_s):
        pv = _vgather(lc_s, prev_idx)
        nx = _vgather(lc_s, next_idx)
        isstart = (iota == 0) | (lc_s != pv)
        isend = (iota == L - 1) | (lc_s != nx)
        startidx = plsc.cummax(jnp.where(isstart, iota, 0))
        off = iota - startidx
        return off, isend

    def hist(j, carry):
        v = idxv[pl.ds(j * L, L)]
        col = v >> 7
        m = (col >= c0) & (col < c1)
        lc = jnp.where(m, col - c0, SENT)
        lc_s, _ = plsc.sort_key_val(lc, v)
        off, isend = runs(lc_s)
        plsc.addupdate_scatter(counts, [lc_s], off + 1,
                               mask=isend & (lc_s < SENT))
        return carry

    lax.fori_loop(0, NV, hist, 0, unroll=4)

    carry = jnp.int32(0)
    for j in range(32):
        cv = counts[pl.ds(j * L, L)]
        cs = plsc.cumsum(cv)
        ex = cs - cv + carry
        bases[pl.ds(j * L, L)] = ex
        bases2[pl.ds(j * L, L)] = ex
        carry = carry + cs[L - 1]

    def place(j, carry2):
        v = idxv[pl.ds(j * L, L)]
        pos = j * L + iota
        col = v >> 7
        m = (col >= c0) & (col < c1)
        lc = jnp.where(m, col - c0, SENT)
        lane7 = v & 127
        packed = pos | (lane7 << 14) | (lc << 21)
        lc_s, packed_s = plsc.sort_key_val(lc, packed)
        off, isend = runs(lc_s)
        mreal = lc_s < SENT
        base_g = plsc.load_gather(bases2, [lc_s])
        plsc.store_scatter(hitbuf, [base_g + off], packed_s, mask=mreal)
        plsc.addupdate_scatter(bases2, [lc_s], off + 1, mask=isend & mreal)
        return carry2

    lax.fori_loop(0, NV, place, 0, unroll=4)

    nt = jnp.int32(0)
    for j in range(32):
        cv = counts[pl.ds(j * L, L)]
        m = cv > 0
        mi = m.astype(jnp.int32)
        within = plsc.cumsum(mi)
        dest = nt + within - mi
        plsc.store_scatter(touched, [dest], j * L + iota, mask=m)
        nt = nt + within[L - 1]

    def issue(t):
        colid = _dx(touched, t)
        fb = (colid + c0) * 128
        slot = lax.rem(t, RING)
        src = T3.at[:, :, pl.ds(pl.multiple_of(fb, 128), 128)]
        pltpu.async_copy(src, panels.at[slot], sem_panel)

    for r in range(RING):
        @pl.when(r < nt)
        def _():
            issue(jnp.int32(r))

    trv = [(c * L + iota) >> 3 for c in range(D // L)]
    fsv = [(c * L + iota) & 7 for c in range(D // L)]

    def loop_t(t, nrows):
        pltpu.make_async_copy(
            T3.at[:, :, pl.ds(0, 128)], panels.at[0], sem_panel).wait()
        slot = lax.rem(t, RING)
        slotv = jnp.full((L,), slot, jnp.int32)
        colid = _dx(touched, t)
        segbase = _dx(bases, colid)
        segcnt = _dx(counts, colid)

        def hloop(h, nr):
            @pl.when(nr >= RSTAGE)
            def _():
                pltpu.make_async_copy(
                    rowstage.at[pl.ds(0, D)], rows_hbm.at[pl.ds(0, D)],
                    sem_row).wait()
            hit = _dx(hitbuf, segbase + h)
            pos = hit & 0x3FFF
            lane7 = (hit >> 14) & 127
            lv = jnp.full((L,), lane7, jnp.int32)
            rslot = lax.rem(nr, RSTAGE)
            for c in range(D // L):
                gv = plsc.load_gather(panels, [slotv, trv[c], fsv[c], lv])
                rowstage[pl.ds(rslot * D + c * L, L)] = gv
            pltpu.async_copy(rowstage.at[pl.ds(rslot * D, D)],
                             rows_hbm.at[pl.ds(pos * D, D)], sem_row)
            return nr + 1

        nrows = lax.fori_loop(0, segcnt, hloop, nrows)

        @pl.when(t + RING < nt)
        def _():
            issue(t + RING)
        return nrows

    nrows = lax.fori_loop(0, nt, loop_t, jnp.int32(0))

    def drain(h, c):
        pltpu.make_async_copy(
            rowstage.at[pl.ds(0, D)], rows_hbm.at[pl.ds(0, D)],
            sem_row).wait()
        return c

    lax.fori_loop(0, jnp.minimum(nrows, RSTAGE), drain, 0)


def _body1(u_hbm, i_hbm, UT_hbm, VT_hbm, urows_hbm, vrows_hbm,
           idxv, hitbuf, counts, bases, bases2, touched, panels, rowstage,
           sem_panel, sem_row):
    wid = lax.axis_index("s") * NC + lax.axis_index("c")
    iota = lax.iota(jnp.int32, L)
    U3 = UT_hbm.reshape(8, 8, UT_hbm.shape[1])
    V3 = VT_hbm.reshape(8, 8, VT_hbm.shape[1])
    half = wid // NS
    hw = wid % NS
    c0 = hw * CPW
    c1 = jnp.minimum(c0 + CPW, COLS)

    @pl.when(half == 0)
    def _():
        _phase(u_hbm, U3, urows_hbm, idxv, hitbuf, counts, bases, bases2,
               touched, panels, rowstage, sem_panel, sem_row, c0, c1, iota)

    @pl.when(half == 1)
    def _():
        _phase(i_hbm, V3, vrows_hbm, idxv, hitbuf, counts, bases, bases2,
               touched, panels, rowstage, sem_panel, sem_row, c0, c1, iota)


def _body2(urows_hbm, vrows_hbm, out_hbm, uv, vv, out_v, sem_u, sem_v):
    bpw = B // NW
    wid = lax.axis_index("s") * NC + lax.axis_index("c")
    base = wid * bpw
    cu = pltpu.async_copy(urows_hbm.at[pl.ds(base * D, bpw * D)], uv, sem_u)
    cv = pltpu.async_copy(vrows_hbm.at[pl.ds(base * D, bpw * D)], vv, sem_v)
    cu.wait()
    cv.wait()
    lane = lax.iota(jnp.int32, L)

    def group(g, carry):
        b0 = g * L
        acc = jnp.zeros((L,), jnp.float32)
        for k in range(L):
            o = (b0 + k) * D
            s = uv[pl.ds(o, L)] * vv[pl.ds(o, L)]
            for c in range(1, D // L):
                s = s + uv[pl.ds(o + c * L, L)] * vv[pl.ds(o + c * L, L)]
            acc = jnp.where(lane == k, jnp.sum(s), acc)
        out_v[pl.ds(b0, L)] = acc
        return carry

    lax.fori_loop(0, B // NW // L, group, 0)
    pltpu.sync_copy(out_v, out_hbm.at[pl.ds(base, bpw)])


def kernel(u, i, U_emb, V_emb):
    mesh = plsc.VectorSubcoreMesh(core_axis_name="c", subcore_axis_name="s")
    f1 = pl.kernel(
        _body1,
        out_type=(jax.ShapeDtypeStruct((B * D,), jnp.float32),
                  jax.ShapeDtypeStruct((B * D,), jnp.float32)),
        mesh=mesh,
        compiler_params=pltpu.CompilerParams(
            needs_layout_passes=False, use_tc_tiling_on_sc=True),
        scratch_types=[
            pltpu.VMEM((B,), jnp.int32),
            pltpu.VMEM((B,), jnp.int32),
            pltpu.VMEM((512,), jnp.int32),
            pltpu.VMEM((512,), jnp.int32),
            pltpu.VMEM((512,), jnp.int32),
            pltpu.VMEM((512,), jnp.int32),
            pltpu.VMEM((RING, 8, 8, 128), jnp.float32),
            pltpu.VMEM((RSTAGE * D,), jnp.float32),
            pltpu.SemaphoreType.DMA,
            pltpu.SemaphoreType.DMA,
        ],
    )
    urows, vrows = f1(u.astype(jnp.int32), i.astype(jnp.int32),
                      U_emb.T, V_emb.T)
    bpw = B // NW
    f2 = pl.kernel(
        _body2,
        out_type=jax.ShapeDtypeStruct((B,), jnp.float32),
        mesh=mesh,
        compiler_params=pltpu.CompilerParams(
            needs_layout_passes=False, use_tc_tiling_on_sc=False),
        scratch_types=[
            pltpu.VMEM((bpw * D,), jnp.float32),
            pltpu.VMEM((bpw * D,), jnp.float32),
            pltpu.VMEM((bpw,), jnp.float32),
            pltpu.SemaphoreType.DMA,
            pltpu.SemaphoreType.DMA,
        ],
    )
    return f2(urows, vrows)


# speedup vs baseline: 1.0693x; 1.0672x over previous
import jax
import jax.numpy as jnp
from jax import lax
from jax.experimental import pallas as pl
from jax.experimental.pallas import tpu as pltpu
from jax.experimental.pallas import tpu_sc as plsc

D = 64
L = 16
NC = 2
NS = 16
NW = NC * NS
B = 16384
NV = B // L
COLS = 7813
CPW = 489
RING = 8
RSTAGE = 32
SENT = 511


_DNUMS = lax.GatherDimensionNumbers(
    offset_dims=(), collapsed_slice_dims=(0,), start_index_map=(0,))


def _vgather(x, idx):
    return lax.gather(x, idx[:, None], _DNUMS, (1,),
                      mode=lax.GatherScatterMode.PROMISE_IN_BOUNDS)


def _dx(ref, t):
    v = ref[pl.ds((t >> 4) << 4, L)]
    g = _vgather(v, jnp.full((L,), t & 15, jnp.int32))
    return g[0]


def _phase(idx_hbm, T3, rows_hbm, idxv, hitbuf, counts, bases, bases2,
           touched, panels, rowstage, sem_panel, sem_row, c0, c1, iota):
    pltpu.sync_copy(idx_hbm, idxv)

    for j in range(32):
        counts[pl.ds(j * L, L)] = jnp.zeros((L,), jnp.int32)

    prev_idx = jnp.maximum(iota - 1, 0)
    next_idx = jnp.minimum(iota + 1, L - 1)

    def runs(lc_s):
        pv = _vgather(lc_s, prev_idx)
        nx = _vgather(lc_s, next_idx)
        isstart = (iota == 0) | (lc_s != pv)
        isend = (iota == L - 1) | (lc_s != nx)
        startidx = plsc.cummax(jnp.where(isstart, iota, 0))
        off = iota - startidx
        return off, isend

    ones = jnp.full((L,), 1, jnp.int32)

    def hist(j, carry):
        v = idxv[pl.ds(j * L, L)]
        col = v >> 7
        m = (col >= c0) & (col < c1)
        lc = jnp.where(m, col - c0, SENT)
        plsc.addupdate_scatter(counts, [lc], ones, mask=m)
        return carry

    lax.fori_loop(0, NV, hist, 0, unroll=4)

    carry = jnp.int32(0)
    for j in range(32):
        cv = counts[pl.ds(j * L, L)]
        cs = plsc.cumsum(cv)
        ex = cs - cv + carry
        bases[pl.ds(j * L, L)] = ex
        bases2[pl.ds(j * L, L)] = ex
        carry = carry + cs[L - 1]

    def place(j, carry2):
        v = idxv[pl.ds(j * L, L)]
        pos = j * L + iota
        col = v >> 7
        m = (col >= c0) & (col < c1)
        lc = jnp.where(m, col - c0, SENT)
        lane7 = v & 127
        packed = pos | (lane7 << 14) | (lc << 21)
        lc_s, packed_s = plsc.sort_key_val(lc, packed)
        off, isend = runs(lc_s)
        mreal = lc_s < SENT
        base_g = plsc.load_gather(bases2, [lc_s])
        plsc.store_scatter(hitbuf, [base_g + off], packed_s, mask=mreal)
        plsc.addupdate_scatter(bases2, [lc_s], off + 1, mask=isend & mreal)
        return carry2

    lax.fori_loop(0, NV, place, 0, unroll=4)

    nt = jnp.int32(0)
    for j in range(32):
        cv = counts[pl.ds(j * L, L)]
        m = cv > 0
        mi = m.astype(jnp.int32)
        within = plsc.cumsum(mi)
        dest = nt + within - mi
        plsc.store_scatter(touched, [dest], j * L + iota, mask=m)
        nt = nt + within[L - 1]

    def issue(t):
        colid = _dx(touched, t)
        fb = (colid + c0) * 128
        slot = lax.rem(t, RING)
        src = T3.at[:, :, pl.ds(pl.multiple_of(fb, 128), 128)]
        pltpu.async_copy(src, panels.at[slot], sem_panel)

    for r in range(RING):
        @pl.when(r < nt)
        def _():
            issue(jnp.int32(r))

    trv = [(c * L + iota) >> 3 for c in range(D // L)]
    fsv = [(c * L + iota) & 7 for c in range(D // L)]

    def loop_t(t, nrows):
        pltpu.make_async_copy(
            T3.at[:, :, pl.ds(0, 128)], panels.at[0], sem_panel).wait()
        slot = lax.rem(t, RING)
        slotv = jnp.full((L,), slot, jnp.int32)
        colid = _dx(touched, t)
        segbase = _dx(bases, colid)
        segcnt = _dx(counts, colid)

        def hloop(h, nr):
            @pl.when(nr >= RSTAGE)
            def _():
                pltpu.make_async_copy(
                    rowstage.at[pl.ds(0, D)], rows_hbm.at[pl.ds(0, D)],
                    sem_row).wait()
            hit = _dx(hitbuf, segbase + h)
            pos = hit & 0x3FFF
            lane7 = (hit >> 14) & 127
            lv = jnp.full((L,), lane7, jnp.int32)
            rslot = lax.rem(nr, RSTAGE)
            for c in range(D // L):
                gv = plsc.load_gather(panels, [slotv, trv[c], fsv[c], lv])
                rowstage[pl.ds(rslot * D + c * L, L)] = gv
            pltpu.async_copy(rowstage.at[pl.ds(rslot * D, D)],
                             rows_hbm.at[pl.ds(pos * D, D)], sem_row)
            return nr + 1

        nrows = lax.fori_loop(0, segcnt, hloop, nrows)

        @pl.when(t + RING < nt)
        def _():
            issue(t + RING)
        return nrows

    nrows = lax.fori_loop(0, nt, loop_t, jnp.int32(0))

    def drain(h, c):
        pltpu.make_async_copy(
            rowstage.at[pl.ds(0, D)], rows_hbm.at[pl.ds(0, D)],
            sem_row).wait()
        return c

    lax.fori_loop(0, jnp.minimum(nrows, RSTAGE), drain, 0)


def _body1(u_hbm, i_hbm, UT_hbm, VT_hbm, urows_hbm, vrows_hbm,
           idxv, hitbuf, counts, bases, bases2, touched, panels, rowstage,
           sem_panel, sem_row):
    wid = lax.axis_index("s") * NC + lax.axis_index("c")
    iota = lax.iota(jnp.int32, L)
    U3 = UT_hbm.reshape(8, 8, UT_hbm.shape[1])
    V3 = VT_hbm.reshape(8, 8, VT_hbm.shape[1])
    half = wid // NS
    hw = wid % NS
    c0 = hw * CPW
    c1 = jnp.minimum(c0 + CPW, COLS)

    @pl.when(half == 0)
    def _():
        _phase(u_hbm, U3, urows_hbm, idxv, hitbuf, counts, bases, bases2,
               touched, panels, rowstage, sem_panel, sem_row, c0, c1, iota)

    @pl.when(half == 1)
    def _():
        _phase(i_hbm, V3, vrows_hbm, idxv, hitbuf, counts, bases, bases2,
               touched, panels, rowstage, sem_panel, sem_row, c0, c1, iota)


def _body2(urows_hbm, vrows_hbm, out_hbm, uv, vv, out_v, sem_u, sem_v):
    bpw = B // NW
    wid = lax.axis_index("s") * NC + lax.axis_index("c")
    base = wid * bpw
    cu = pltpu.async_copy(urows_hbm.at[pl.ds(base * D, bpw * D)], uv, sem_u)
    cv = pltpu.async_copy(vrows_hbm.at[pl.ds(base * D, bpw * D)], vv, sem_v)
    cu.wait()
    cv.wait()
    lane = lax.iota(jnp.int32, L)

    def group(g, carry):
        b0 = g * L
        acc = jnp.zeros((L,), jnp.float32)
        for k in range(L):
            o = (b0 + k) * D
            s = uv[pl.ds(o, L)] * vv[pl.ds(o, L)]
            for c in range(1, D // L):
                s = s + uv[pl.ds(o + c * L, L)] * vv[pl.ds(o + c * L, L)]
            acc = jnp.where(lane == k, jnp.sum(s), acc)
        out_v[pl.ds(b0, L)] = acc
        return carry

    lax.fori_loop(0, B // NW // L, group, 0)
    pltpu.sync_copy(out_v, out_hbm.at[pl.ds(base, bpw)])


def kernel(u, i, U_emb, V_emb):
    mesh = plsc.VectorSubcoreMesh(core_axis_name="c", subcore_axis_name="s")
    f1 = pl.kernel(
        _body1,
        out_type=(jax.ShapeDtypeStruct((B * D,), jnp.float32),
                  jax.ShapeDtypeStruct((B * D,), jnp.float32)),
        mesh=mesh,
        compiler_params=pltpu.CompilerParams(
            needs_layout_passes=False, use_tc_tiling_on_sc=True),
        scratch_types=[
            pltpu.VMEM((B,), jnp.int32),
            pltpu.VMEM((B,), jnp.int32),
            pltpu.VMEM((512,), jnp.int32),
            pltpu.VMEM((512,), jnp.int32),
            pltpu.VMEM((512,), jnp.int32),
            pltpu.VMEM((512,), jnp.int32),
            pltpu.VMEM((RING, 8, 8, 128), jnp.float32),
            pltpu.VMEM((RSTAGE * D,), jnp.float32),
            pltpu.SemaphoreType.DMA,
            pltpu.SemaphoreType.DMA,
        ],
    )
    urows, vrows = f1(u.astype(jnp.int32), i.astype(jnp.int32),
                      U_emb.T, V_emb.T)
    bpw = B // NW
    f2 = pl.kernel(
        _body2,
        out_type=jax.ShapeDtypeStruct((B,), jnp.float32),
        mesh=mesh,
        compiler_params=pltpu.CompilerParams(
            needs_layout_passes=False, use_tc_tiling_on_sc=False),
        scratch_types=[
            pltpu.VMEM((bpw * D,), jnp.float32),
            pltpu.VMEM((bpw * D,), jnp.float32),
            pltpu.VMEM((bpw,), jnp.float32),
            pltpu.SemaphoreType.DMA,
            pltpu.SemaphoreType.DMA,
        ],
    )
    return f2(urows, vrows)
